# baseline (speedup 1.0000x reference)
import functools
from functools import partial

import jax
import jax.numpy as jnp
from jax.experimental import pallas as pl
from jax.experimental.pallas import tpu as pltpu
from jax.experimental.pallas import tpu_sc as plsc

_CFG = [
    (1024, 0.05, 20, 32, 64, 2),
    (1024, 0.05, 20, 64, 64, 4),
    (1024, 0.05, 20, 64, 128, 2),
    (1024, 0.1, 20, 128, 128, 4),
    (256, 0.1, 20, 128, 256, 2),
    (256, 0.2, 20, 256, 256, 4),
    (64, 0.2, 20, 256, 512, 2),
    (64, 0.4, 20, 512, 512, 4),
]


def _leaky(x):
    return jnp.where(x >= 0, x, 0.01 * x)


def _dot(a, b):
    return jnp.dot(a, b, preferred_element_type=jnp.float32)


def _fps_body(px_ref, py_ref, pz_ref, pce_ref, *, n, N, B):
    S = N // 128
    px = px_ref[...]
    py = py_ref[...]
    pz = pz_ref[...]
    row = jax.lax.broadcasted_iota(jnp.int32, (B, S, 128), 1)
    col = jax.lax.broadcasted_iota(jnp.int32, (B, S, 128), 2)
    flat = row * 128 + col

    def body(i, st):
        dist, far = st
        sel = flat == far
        cx = jnp.sum(jnp.where(sel, px, 0.0), axis=(1, 2), keepdims=True)
        cy = jnp.sum(jnp.where(sel, py, 0.0), axis=(1, 2), keepdims=True)
        cz = jnp.sum(jnp.where(sel, pz, 0.0), axis=(1, 2), keepdims=True)
        c = jnp.concatenate([cx, cy, cz], axis=2)
        pce_ref[:, pl.ds(i, 1), :] = c
        d = (px - cx) ** 2 + (py - cy) ** 2 + (pz - cz) ** 2
        dist = jnp.minimum(dist, d)
        mv = jnp.max(dist, axis=(1, 2), keepdims=True)
        far = jnp.min(jnp.where(dist == mv, flat, N), axis=(1, 2),
                      keepdims=True)
        return dist, far

    jax.lax.fori_loop(
        0, n, body,
        (jnp.full((B, S, 128), 1e10, jnp.float32),
         jnp.zeros((B, 1, 1), jnp.int32)))


def _fps(p, n):
    B, N, _ = p.shape
    S = N // 128
    pp = p.transpose(0, 2, 1).reshape(B, 3, S, 128)
    px, py, pz = pp[:, 0], pp[:, 1], pp[:, 2]
    full = lambda shape: pl.BlockSpec(shape, lambda g: tuple(0 for _ in shape))
    return pl.pallas_call(
        partial(_fps_body, n=n, N=N, B=B),
        grid=(1,),
        in_specs=[full((B, S, 128)), full((B, S, 128)), full((B, S, 128))],
        out_specs=full((B, n, 3)),
        out_shape=jax.ShapeDtypeStruct((B, n, 3), jnp.float32),
    )(px, py, pz)


def _ball_body(pq_ref, pkt_ref, f_ref, o_ref, *, k, r2, N):
    q = pq_ref[0]
    pk = pkt_ref[0]
    qq = jnp.sum(q * q, axis=1, keepdims=True)
    x, y, z = pk[0:1, :], pk[1:2, :], pk[2:3, :]
    pp = (x * x + y * y) + z * z
    cross = jnp.dot(q, pk, preferred_element_type=jnp.float32)
    sqr = (qq + pp) - 2.0 * cross
    Mt = q.shape[0]
    iota_n = jax.lax.broadcasted_iota(jnp.int32, (Mt, N), 1)
    key = jnp.where(sqr <= r2, iota_n, N)
    f2 = f_ref[0]
    G2 = f2.shape[0]
    P = N // G2
    D = f2.shape[1] // P
    iota_g = jax.lax.broadcasted_iota(jnp.int32, (Mt, G2), 1)
    iota_d = jax.lax.broadcasted_iota(jnp.int32, (Mt, D), 1)
    acc = jnp.full((Mt, D), -jnp.inf, jnp.float32)
    empty = None
    for _ in range(k):
        v = jnp.min(key, axis=1, keepdims=True)
        if empty is None:
            empty = v >= N
        oh = (iota_g == v // P).astype(jnp.float32)
        wide = _dot(oh, f2)
        if P > 1:
            lane = (v % P) * D + iota_d
            fj = jnp.take_along_axis(wide, lane, axis=1)
        else:
            fj = wide
        acc = jnp.maximum(acc, jnp.where(v < N, fj, -jnp.inf))
        key = jnp.where(key == v, N, key)
    f_last = f2[G2 - 1:G2, (P - 1) * D:]
    o_ref[0] = jnp.where(empty, f_last, acc)


def _ball(p, pce, f, k, r):
    B, N, _ = p.shape
    M = pce.shape[1]
    D = f.shape[2]
    Mt = min(M, 128)
    P = max(1, 128 // D)
    f2 = f.reshape(B, N // P, P * D)
    pkt = p.transpose(0, 2, 1)
    return pl.pallas_call(
        partial(_ball_body, k=k, r2=r * r, N=N),
        grid=(B, M // Mt),
        in_specs=[pl.BlockSpec((1, Mt, 3), lambda b, m: (b, m, 0)),
                  pl.BlockSpec((1, 3, N), lambda b, m: (b, 0, 0)),
                  pl.BlockSpec((1, N // P, P * D), lambda b, m: (b, 0, 0))],
        out_specs=pl.BlockSpec((1, Mt, D), lambda b, m: (b, m, 0)),
        out_shape=jax.ShapeDtypeStruct((B, M, D), jnp.float32),
    )(pce, pkt, f2)


def _knn_body(pq_ref, pkt_ref, idx_ref, *, k, M):
    q = pq_ref[0]
    pk = pkt_ref[0]
    qq = jnp.sum(q * q, axis=1, keepdims=True)
    x, y, z = pk[0:1, :], pk[1:2, :], pk[2:3, :]
    pp = (x * x + y * y) + z * z
    cross = jnp.dot(q, pk, preferred_element_type=jnp.float32)
    key = (qq + pp) - 2.0 * cross
    Mt = q.shape[0]
    iota_m = jax.lax.broadcasted_iota(jnp.int32, (Mt, M), 1)
    cols = []
    for _ in range(k):
        v = jnp.min(key, axis=1, keepdims=True)
        nj = jnp.min(jnp.where(key == v, iota_m, M), axis=1, keepdims=True)
        cols.append(nj)
        key = jnp.where(iota_m == nj, jnp.inf, key)
    idx_ref[0] = jnp.concatenate(cols, axis=1)


def _knn(p, k):
    B, M, _ = p.shape
    Mt = min(M, 128)
    pkt = p.transpose(0, 2, 1)
    return pl.pallas_call(
        partial(_knn_body, k=k, M=M),
        grid=(B, M // Mt),
        in_specs=[pl.BlockSpec((1, Mt, 3), lambda b, m: (b, m, 0)),
                  pl.BlockSpec((1, 3, M), lambda b, m: (b, 0, 0))],
        out_specs=pl.BlockSpec((1, Mt, k), lambda b, m: (b, m, 0)),
        out_shape=jax.ShapeDtypeStruct((B, M, k), jnp.int32),
    )(p, pkt)


def _prep_body(f_ref, p_ref, preW, preb, A, Bw, p2fb, s_ref, t_ref):
    f = f_ref[0]
    p_ = p_ref[0]
    fpre = _leaky(_dot(f, preW[...]) + preb[...])
    s_ref[0] = fpre + _dot(p_, Bw[...])
    t_ref[0] = _dot(p_, A[...]) + p2fb[...] - fpre


def _prep_sc_body(f_ref, p_ref, preW, preb, A, Bw, p2fb, scW, scb,
                  s_ref, t_ref, res_ref):
    f = f_ref[0]
    p_ = p_ref[0]
    fpre = _leaky(_dot(f, preW[...]) + preb[...])
    s_ref[0] = fpre + _dot(p_, Bw[...])
    t_ref[0] = _dot(p_, A[...]) + p2fb[...] - fpre
    res_ref[0] = _dot(f, scW[...]) + scb[...]


def _prep(f, pts, pr):
    B, M, d_in = f.shape
    dh = pr['preW'].shape[1]
    p2fW = pr['p2fW']
    A = p2fW[0:3] - p2fW[6:9]
    Bw = p2fW[3:6] + p2fW[6:9]
    preb = pr['preb'].reshape(1, dh)
    p2fb = pr['p2fb'].reshape(1, dh)
    wspec = lambda shape: pl.BlockSpec(shape, lambda b: tuple(0 for _ in shape))
    has_sc = 'scW' in pr
    if has_sc:
        d_out = pr['scW'].shape[1]
        scb = pr['scb'].reshape(1, d_out)
        out_shapes = [jax.ShapeDtypeStruct((B, M, dh), jnp.float32),
                      jax.ShapeDtypeStruct((B, M, dh), jnp.float32),
                      jax.ShapeDtypeStruct((B, M, d_out), jnp.float32)]
        out_specs = [pl.BlockSpec((1, M, dh), lambda b: (b, 0, 0)),
                     pl.BlockSpec((1, M, dh), lambda b: (b, 0, 0)),
                     pl.BlockSpec((1, M, d_out), lambda b: (b, 0, 0))]
        s, t, res = pl.pallas_call(
            _prep_sc_body,
            grid=(B,),
            in_specs=[pl.BlockSpec((1, M, d_in), lambda b: (b, 0, 0)),
                      pl.BlockSpec((1, M, 3), lambda b: (b, 0, 0)),
                      wspec((d_in, dh)), wspec((1, dh)),
                      wspec((3, dh)), wspec((3, dh)), wspec((1, dh)),
                      wspec((d_in, d_out)), wspec((1, d_out))],
            out_specs=out_specs,
            out_shape=out_shapes,
        )(f, pts, pr['preW'], preb, A, Bw, p2fb, pr['scW'], scb)
        return s, t, res
    s, t = pl.pallas_call(
        _prep_body,
        grid=(B,),
        in_specs=[pl.BlockSpec((1, M, d_in), lambda b: (b, 0, 0)),
                  pl.BlockSpec((1, M, 3), lambda b: (b, 0, 0)),
                  wspec((d_in, dh)), wspec((1, dh)),
                  wspec((3, dh)), wspec((3, dh)), wspec((1, dh))],
        out_specs=[pl.BlockSpec((1, M, dh), lambda b: (b, 0, 0)),
                   pl.BlockSpec((1, M, dh), lambda b: (b, 0, 0))],
        out_shape=[jax.ShapeDtypeStruct((B, M, dh), jnp.float32),
                   jax.ShapeDtypeStruct((B, M, dh), jnp.float32)],
    )(f, pts, pr['preW'], preb, A, Bw, p2fb)
    return s, t, f


def _sc_gather(s_flat, gidx, dh):
    G = gidx.shape[1]
    window = 128
    mesh = plsc.VectorSubcoreMesh(core_axis_name="core",
                                  subcore_axis_name="subcore")

    @partial(pl.kernel,
             out_type=jax.ShapeDtypeStruct((G, dh), jnp.float32),
             mesh=mesh)
    def kern(x_hbm, i_hbm, o_hbm):
        def body(i_vmem, o_vmem):
            pltpu.sync_copy(x_hbm.at[i_vmem.at[0]], o_vmem)

        pltpu.emit_pipeline(
            body,
            grid=(G // window,),
            in_specs=[pl.BlockSpec((1, window), lambda i: (0, i))],
            out_specs=[pl.BlockSpec((window, dh), lambda i: (i, 0))],
            core_axis_name=("core", "subcore"),
            dimension_semantics=(pltpu.PARALLEL,),
        )(i_hbm, o_hbm)

    return kern(s_flat, gidx)


def _core_body(s2_ref, t_ref, idx_ref, res_ref, mlpW, mlpb, pstW, pstb,
               o_ref, *, k, P, dh):
    s2 = s2_ref[0]
    t = t_ref[0]
    idx = idx_ref[0]
    Mt = t.shape[0]
    G2 = s2.shape[0]
    iota_g = jax.lax.broadcasted_iota(jnp.int32, (Mt, G2), 1)
    iota_d = jax.lax.broadcasted_iota(jnp.int32, (Mt, dh), 1)
    acc = None
    w = mlpW[...]
    b = mlpb[...]
    for j in range(k):
        ij = idx[:, j:j + 1]
        oh = (iota_g == ij // P).astype(jnp.float32)
        wide = _dot(oh, s2)
        if P > 1:
            lane = (ij % P) * dh + iota_d
            sj = jnp.take_along_axis(wide, lane, axis=1)
        else:
            sj = wide
        g1 = _leaky(sj + t)
        g2 = _leaky(_dot(g1, w) + b)
        acc = g2 if acc is None else jnp.maximum(acc, g2)
    o_ref[0] = _leaky(_dot(acc, pstW[...]) + pstb[...] + res_ref[0])


def _core(s, t, idx, res, pr, k):
    B, M, dh = s.shape
    d_out = pr['pstW'].shape[1]
    Mt = min(M, 128)
    P = max(1, 128 // dh)
    s2 = s.reshape(B, M // P, P * dh)
    mlpb = pr['mlpb'].reshape(1, dh)
    pstb = pr['pstb'].reshape(1, d_out)
    wspec = lambda shape: pl.BlockSpec(shape, lambda b, m: tuple(0 for _ in shape))
    return pl.pallas_call(
        partial(_core_body, k=k, P=P, dh=dh),
        grid=(B, M // Mt),
        in_specs=[pl.BlockSpec((1, M // P, P * dh), lambda b, m: (b, 0, 0)),
                  pl.BlockSpec((1, Mt, dh), lambda b, m: (b, m, 0)),
                  pl.BlockSpec((1, Mt, k), lambda b, m: (b, m, 0)),
                  pl.BlockSpec((1, Mt, d_out), lambda b, m: (b, m, 0)),
                  wspec((dh, dh)), wspec((1, dh)),
                  wspec((dh, d_out)), wspec((1, d_out))],
        out_specs=pl.BlockSpec((1, Mt, d_out), lambda b, m: (b, m, 0)),
        out_shape=jax.ShapeDtypeStruct((B, M, d_out), jnp.float32),
    )(s2, t, idx, res, pr['mlpW'], mlpb, pr['pstW'], pstb)


def _core_sc_body(sg_ref, t_ref, res_ref, mlpW, mlpb, pstW, pstb,
                  o_ref, *, k):
    t = t_ref[0]
    acc = None
    w = mlpW[...]
    b = mlpb[...]
    for j in range(k):
        sj = sg_ref[0, j]
        g1 = _leaky(sj + t)
        g2 = _leaky(_dot(g1, w) + b)
        acc = g2 if acc is None else jnp.maximum(acc, g2)
    o_ref[0] = _leaky(_dot(acc, pstW[...]) + pstb[...] + res_ref[0])


def _core_sc(sg, t, res, pr, k):
    B, _, M, dh = sg.shape
    d_out = pr['pstW'].shape[1]
    Mt = min(M, 128)
    mlpb = pr['mlpb'].reshape(1, dh)
    pstb = pr['pstb'].reshape(1, d_out)
    wspec = lambda shape: pl.BlockSpec(shape, lambda b, m: tuple(0 for _ in shape))
    return pl.pallas_call(
        partial(_core_sc_body, k=k),
        grid=(B, M // Mt),
        in_specs=[pl.BlockSpec((1, k, Mt, dh), lambda b, m: (b, 0, m, 0)),
                  pl.BlockSpec((1, Mt, dh), lambda b, m: (b, m, 0)),
                  pl.BlockSpec((1, Mt, d_out), lambda b, m: (b, m, 0)),
                  wspec((dh, dh)), wspec((1, dh)),
                  wspec((dh, d_out)), wspec((1, d_out))],
        out_specs=pl.BlockSpec((1, Mt, d_out), lambda b, m: (b, m, 0)),
        out_shape=jax.ShapeDtypeStruct((B, M, d_out), jnp.float32),
    )(sg, t, res, pr['mlpW'], mlpb, pr['pstW'], pstb)


def _stem_body(p_ref, W, b, o_ref):
    o_ref[0] = _leaky(_dot(p_ref[0], W[...]) + b[...])


def _stem(p, W, b):
    B, N, _ = p.shape
    D = W.shape[1]
    return pl.pallas_call(
        _stem_body,
        grid=(B,),
        in_specs=[pl.BlockSpec((1, N, 3), lambda i: (i, 0, 0)),
                  pl.BlockSpec((3, D), lambda i: (0, 0)),
                  pl.BlockSpec((1, D), lambda i: (0, 0))],
        out_specs=pl.BlockSpec((1, N, D), lambda i: (i, 0, 0)),
        out_shape=jax.ShapeDtypeStruct((B, N, D), jnp.float32),
    )(p, W, b.reshape(1, D))


def _head_body(f_ref, h1W, h1b, h2W, h2b, h3W, h3b, o_ref):
    f = f_ref[0]
    x = jnp.maximum(_dot(f, h1W[...]) + h1b[...], 0.0)
    mx = jnp.max(x, axis=0, keepdims=True)
    mn = jnp.mean(x, axis=0, keepdims=True)
    xc = jnp.concatenate([mx, mn], axis=1)
    x2 = jnp.maximum(_dot(xc, h2W[...]) + h2b[...], 0.0)
    o_ref[0] = _dot(x2, h3W[...]) + h3b[...]


def _head(f, hd):
    B, M, D = f.shape
    wspec = lambda shape: pl.BlockSpec(shape, lambda b: tuple(0 for _ in shape))
    out = pl.pallas_call(
        _head_body,
        grid=(B,),
        in_specs=[pl.BlockSpec((1, M, D), lambda b: (b, 0, 0)),
                  wspec((D, 1024)), wspec((1, 1024)),
                  wspec((2048, 512)), wspec((1, 512)),
                  wspec((512, 40)), wspec((1, 40))],
        out_specs=pl.BlockSpec((1, 1, 40), lambda b: (b, 0, 0)),
        out_shape=jax.ShapeDtypeStruct((B, 1, 40), jnp.float32),
    )(f, hd['h1W'], hd['h1b'].reshape(1, 1024), hd['h2W'],
      hd['h2b'].reshape(1, 512), hd['h3W'], hd['h3b'].reshape(1, 40))
    return out.reshape(B, 40)


def kernel(p, params):
    B = p.shape[0]
    f = _stem(p, params['stem']['W'], params['stem']['b'])
    pts = p
    idx = None
    gidx = None
    for pr, cfg in zip(params['cic'], _CFG):
        n_g, r_g, k_g, d_in, d_out, ratio = cfg
        M = n_g
        if pts.shape[1] != n_g:
            pce = _fps(pts, n_g)
            f = _ball(pts, pce, f, k_g, r_g)
            pts = pce
            idx = _knn(pts, k_g)
            gidx = (idx.transpose(0, 2, 1)
                    + (jnp.arange(B, dtype=jnp.int32) * M).reshape(B, 1, 1))
            gidx = gidx.reshape(1, B * k_g * M)
        s, t, res = _prep(f, pts, pr)
        dh = s.shape[2]
        if dh == 128:
            sg = _sc_gather(s.reshape(B * M, dh), gidx, dh)
            f = _core_sc(sg.reshape(B, k_g, M, dh), t, res, pr, k_g)
        else:
            f = _core(s, t, idx, res, pr, k_g)
    return _head(f, params['head'])


# speedup vs baseline: 1.2817x; 1.2817x over previous
import functools
from functools import partial

import jax
import jax.numpy as jnp
from jax.experimental import pallas as pl
from jax.experimental.pallas import tpu as pltpu
from jax.experimental.pallas import tpu_sc as plsc

_CFG = [
    (1024, 0.05, 20, 32, 64, 2),
    (1024, 0.05, 20, 64, 64, 4),
    (1024, 0.05, 20, 64, 128, 2),
    (1024, 0.1, 20, 128, 128, 4),
    (256, 0.1, 20, 128, 256, 2),
    (256, 0.2, 20, 256, 256, 4),
    (64, 0.2, 20, 256, 512, 2),
    (64, 0.4, 20, 512, 512, 4),
]


def _leaky(x):
    return jnp.where(x >= 0, x, 0.01 * x)


def _dot(a, b):
    return jnp.dot(a, b, preferred_element_type=jnp.float32)


def _fps_body(px_ref, py_ref, pz_ref, pce_ref, *, n, N, B):
    S = N // 128
    px = px_ref[...]
    py = py_ref[...]
    pz = pz_ref[...]
    row = jax.lax.broadcasted_iota(jnp.int32, (B, S, 128), 1)
    col = jax.lax.broadcasted_iota(jnp.int32, (B, S, 128), 2)
    flat = row * 128 + col

    def body(i, st):
        dist, far = st
        sel = flat == far
        cx = jnp.sum(jnp.where(sel, px, 0.0), axis=(1, 2), keepdims=True)
        cy = jnp.sum(jnp.where(sel, py, 0.0), axis=(1, 2), keepdims=True)
        cz = jnp.sum(jnp.where(sel, pz, 0.0), axis=(1, 2), keepdims=True)
        c = jnp.concatenate([cx, cy, cz], axis=2)
        pce_ref[:, pl.ds(i, 1), :] = c
        d = (px - cx) ** 2 + (py - cy) ** 2 + (pz - cz) ** 2
        dist = jnp.minimum(dist, d)
        mv = jnp.max(dist, axis=(1, 2), keepdims=True)
        far = jnp.min(jnp.where(dist == mv, flat, N), axis=(1, 2),
                      keepdims=True)
        return dist, far

    jax.lax.fori_loop(
        0, n, body,
        (jnp.full((B, S, 128), 1e10, jnp.float32),
         jnp.zeros((B, 1, 1), jnp.int32)))


def _fps(p, n):
    B, N, _ = p.shape
    S = N // 128
    pp = p.transpose(0, 2, 1).reshape(B, 3, S, 128)
    px, py, pz = pp[:, 0], pp[:, 1], pp[:, 2]
    full = lambda shape: pl.BlockSpec(shape, lambda g: tuple(0 for _ in shape))
    return pl.pallas_call(
        partial(_fps_body, n=n, N=N, B=B),
        grid=(1,),
        in_specs=[full((B, S, 128)), full((B, S, 128)), full((B, S, 128))],
        out_specs=full((B, n, 3)),
        out_shape=jax.ShapeDtypeStruct((B, n, 3), jnp.float32),
    )(px, py, pz)


def _ball_body(pq_ref, pkt_ref, f_ref, o_ref, *, k, r2, N):
    q = pq_ref[0]
    pk = pkt_ref[0]
    qq = jnp.sum(q * q, axis=1, keepdims=True)
    x, y, z = pk[0:1, :], pk[1:2, :], pk[2:3, :]
    pp = (x * x + y * y) + z * z
    cross = jnp.dot(q, pk, preferred_element_type=jnp.float32)
    sqr = (qq + pp) - 2.0 * cross
    Mt = q.shape[0]
    iota_n = jax.lax.broadcasted_iota(jnp.int32, (Mt, N), 1)
    key = jnp.where(sqr <= r2, iota_n, N)
    f2 = f_ref[0]
    G2 = f2.shape[0]
    P = N // G2
    D = f2.shape[1] // P
    iota_g = jax.lax.broadcasted_iota(jnp.int32, (Mt, G2), 1)
    iota_d = jax.lax.broadcasted_iota(jnp.int32, (Mt, D), 1)
    acc = jnp.full((Mt, D), -jnp.inf, jnp.float32)
    v0 = jnp.min(key, axis=1, keepdims=True)
    empty = v0 >= N

    def step(st):
        j, key, acc, v = st
        oh = (iota_g == v // P).astype(jnp.float32)
        wide = _dot(oh, f2)
        if P > 1:
            lane = (v % P) * D + iota_d
            fj = jnp.take_along_axis(wide, lane, axis=1)
        else:
            fj = wide
        acc = jnp.maximum(acc, jnp.where(v < N, fj, -jnp.inf))
        key = jnp.where(key == v, N, key)
        v = jnp.min(key, axis=1, keepdims=True)
        return j + 1, key, acc, v

    def cond(st):
        j, _, _, v = st
        return jnp.logical_and(j < k, jnp.min(v) < N)

    _, _, acc, _ = jax.lax.while_loop(
        cond, step, (jnp.int32(0), key, acc, v0))
    f_last = f2[G2 - 1:G2, (P - 1) * D:]
    o_ref[0] = jnp.where(empty, f_last, acc)


def _ball(p, pce, f, k, r):
    B, N, _ = p.shape
    M = pce.shape[1]
    D = f.shape[2]
    Mt = min(M, 256)
    P = max(1, 128 // D)
    f2 = f.reshape(B, N // P, P * D)
    pkt = p.transpose(0, 2, 1)
    return pl.pallas_call(
        partial(_ball_body, k=k, r2=r * r, N=N),
        grid=(B, M // Mt),
        in_specs=[pl.BlockSpec((1, Mt, 3), lambda b, m: (b, m, 0)),
                  pl.BlockSpec((1, 3, N), lambda b, m: (b, 0, 0)),
                  pl.BlockSpec((1, N // P, P * D), lambda b, m: (b, 0, 0))],
        out_specs=pl.BlockSpec((1, Mt, D), lambda b, m: (b, m, 0)),
        out_shape=jax.ShapeDtypeStruct((B, M, D), jnp.float32),
    )(pce, pkt, f2)


def _knn_body(pq_ref, pkt_ref, idx_ref, *, k, M):
    q = pq_ref[0]
    pk = pkt_ref[0]
    qq = jnp.sum(q * q, axis=1, keepdims=True)
    x, y, z = pk[0:1, :], pk[1:2, :], pk[2:3, :]
    pp = (x * x + y * y) + z * z
    cross = jnp.dot(q, pk, preferred_element_type=jnp.float32)
    key = (qq + pp) - 2.0 * cross
    Mt = q.shape[0]
    iota_m = jax.lax.broadcasted_iota(jnp.int32, (Mt, M), 1)
    cols = []
    for _ in range(k):
        v = jnp.min(key, axis=1, keepdims=True)
        nj = jnp.min(jnp.where(key == v, iota_m, M), axis=1, keepdims=True)
        cols.append(nj)
        key = jnp.where(iota_m == nj, jnp.inf, key)
    idx_ref[0] = jnp.concatenate(cols, axis=1)


def _knn(p, k):
    B, M, _ = p.shape
    Mt = min(M, 256)
    pkt = p.transpose(0, 2, 1)
    return pl.pallas_call(
        partial(_knn_body, k=k, M=M),
        grid=(B, M // Mt),
        in_specs=[pl.BlockSpec((1, Mt, 3), lambda b, m: (b, m, 0)),
                  pl.BlockSpec((1, 3, M), lambda b, m: (b, 0, 0))],
        out_specs=pl.BlockSpec((1, Mt, k), lambda b, m: (b, m, 0)),
        out_shape=jax.ShapeDtypeStruct((B, M, k), jnp.int32),
    )(p, pkt)


def _prep_body(f_ref, p_ref, preW, preb, A, Bw, p2fb, s_ref, t_ref):
    f = f_ref[0]
    p_ = p_ref[0]
    fpre = _leaky(_dot(f, preW[...]) + preb[...])
    s_ref[0] = fpre + _dot(p_, Bw[...])
    t_ref[0] = _dot(p_, A[...]) + p2fb[...] - fpre


def _prep_sc_body(f_ref, p_ref, preW, preb, A, Bw, p2fb, scW, scb,
                  s_ref, t_ref, res_ref):
    f = f_ref[0]
    p_ = p_ref[0]
    fpre = _leaky(_dot(f, preW[...]) + preb[...])
    s_ref[0] = fpre + _dot(p_, Bw[...])
    t_ref[0] = _dot(p_, A[...]) + p2fb[...] - fpre
    res_ref[0] = _dot(f, scW[...]) + scb[...]


def _prep(f, pts, pr):
    B, M, d_in = f.shape
    dh = pr['preW'].shape[1]
    p2fW = pr['p2fW']
    A = p2fW[0:3] - p2fW[6:9]
    Bw = p2fW[3:6] + p2fW[6:9]
    preb = pr['preb'].reshape(1, dh)
    p2fb = pr['p2fb'].reshape(1, dh)
    wspec = lambda shape: pl.BlockSpec(shape, lambda b: tuple(0 for _ in shape))
    has_sc = 'scW' in pr
    if has_sc:
        d_out = pr['scW'].shape[1]
        scb = pr['scb'].reshape(1, d_out)
        out_shapes = [jax.ShapeDtypeStruct((B, M, dh), jnp.float32),
                      jax.ShapeDtypeStruct((B, M, dh), jnp.float32),
                      jax.ShapeDtypeStruct((B, M, d_out), jnp.float32)]
        out_specs = [pl.BlockSpec((1, M, dh), lambda b: (b, 0, 0)),
                     pl.BlockSpec((1, M, dh), lambda b: (b, 0, 0)),
                     pl.BlockSpec((1, M, d_out), lambda b: (b, 0, 0))]
        s, t, res = pl.pallas_call(
            _prep_sc_body,
            grid=(B,),
            in_specs=[pl.BlockSpec((1, M, d_in), lambda b: (b, 0, 0)),
                      pl.BlockSpec((1, M, 3), lambda b: (b, 0, 0)),
                      wspec((d_in, dh)), wspec((1, dh)),
                      wspec((3, dh)), wspec((3, dh)), wspec((1, dh)),
                      wspec((d_in, d_out)), wspec((1, d_out))],
            out_specs=out_specs,
            out_shape=out_shapes,
        )(f, pts, pr['preW'], preb, A, Bw, p2fb, pr['scW'], scb)
        return s, t, res
    s, t = pl.pallas_call(
        _prep_body,
        grid=(B,),
        in_specs=[pl.BlockSpec((1, M, d_in), lambda b: (b, 0, 0)),
                  pl.BlockSpec((1, M, 3), lambda b: (b, 0, 0)),
                  wspec((d_in, dh)), wspec((1, dh)),
                  wspec((3, dh)), wspec((3, dh)), wspec((1, dh))],
        out_specs=[pl.BlockSpec((1, M, dh), lambda b: (b, 0, 0)),
                   pl.BlockSpec((1, M, dh), lambda b: (b, 0, 0))],
        out_shape=[jax.ShapeDtypeStruct((B, M, dh), jnp.float32),
                   jax.ShapeDtypeStruct((B, M, dh), jnp.float32)],
    )(f, pts, pr['preW'], preb, A, Bw, p2fb)
    return s, t, f


def _sc_gather(s_flat, gidx, dh):
    G = gidx.shape[1]
    window = 128
    mesh = plsc.VectorSubcoreMesh(core_axis_name="core",
                                  subcore_axis_name="subcore")

    @partial(pl.kernel,
             out_type=jax.ShapeDtypeStruct((G, dh), jnp.float32),
             mesh=mesh)
    def kern(x_hbm, i_hbm, o_hbm):
        def body(i_vmem, o_vmem):
            pltpu.sync_copy(x_hbm.at[i_vmem.at[0]], o_vmem)

        pltpu.emit_pipeline(
            body,
            grid=(G // window,),
            in_specs=[pl.BlockSpec((1, window), lambda i: (0, i))],
            out_specs=[pl.BlockSpec((window, dh), lambda i: (i, 0))],
            core_axis_name=("core", "subcore"),
            dimension_semantics=(pltpu.PARALLEL,),
        )(i_hbm, o_hbm)

    return kern(s_flat, gidx)


def _core_body(s2_ref, t_ref, idx_ref, res_ref, mlpW, mlpb, pstW, pstb,
               o_ref, *, k, P, dh):
    s2 = s2_ref[0]
    t = t_ref[0]
    idx = idx_ref[0]
    Mt = t.shape[0]
    G2 = s2.shape[0]
    iota_g = jax.lax.broadcasted_iota(jnp.int32, (Mt, G2), 1)
    iota_d = jax.lax.broadcasted_iota(jnp.int32, (Mt, dh), 1)
    acc = None
    w = mlpW[...]
    b = mlpb[...]
    for j in range(k):
        ij = idx[:, j:j + 1]
        oh = (iota_g == ij // P).astype(jnp.float32)
        wide = _dot(oh, s2)
        if P > 1:
            lane = (ij % P) * dh + iota_d
            sj = jnp.take_along_axis(wide, lane, axis=1)
        else:
            sj = wide
        g1 = _leaky(sj + t)
        g2 = _leaky(_dot(g1, w) + b)
        acc = g2 if acc is None else jnp.maximum(acc, g2)
    o_ref[0] = _leaky(_dot(acc, pstW[...]) + pstb[...] + res_ref[0])


def _core(s, t, idx, res, pr, k):
    B, M, dh = s.shape
    d_out = pr['pstW'].shape[1]
    Mt = min(M, 256)
    P = max(1, 128 // dh)
    s2 = s.reshape(B, M // P, P * dh)
    mlpb = pr['mlpb'].reshape(1, dh)
    pstb = pr['pstb'].reshape(1, d_out)
    wspec = lambda shape: pl.BlockSpec(shape, lambda b, m: tuple(0 for _ in shape))
    return pl.pallas_call(
        partial(_core_body, k=k, P=P, dh=dh),
        grid=(B, M // Mt),
        in_specs=[pl.BlockSpec((1, M // P, P * dh), lambda b, m: (b, 0, 0)),
                  pl.BlockSpec((1, Mt, dh), lambda b, m: (b, m, 0)),
                  pl.BlockSpec((1, Mt, k), lambda b, m: (b, m, 0)),
                  pl.BlockSpec((1, Mt, d_out), lambda b, m: (b, m, 0)),
                  wspec((dh, dh)), wspec((1, dh)),
                  wspec((dh, d_out)), wspec((1, d_out))],
        out_specs=pl.BlockSpec((1, Mt, d_out), lambda b, m: (b, m, 0)),
        out_shape=jax.ShapeDtypeStruct((B, M, d_out), jnp.float32),
    )(s2, t, idx, res, pr['mlpW'], mlpb, pr['pstW'], pstb)


def _core_sc_body(sg_ref, t_ref, res_ref, mlpW, mlpb, pstW, pstb,
                  o_ref, *, k):
    t = t_ref[0]
    acc = None
    w = mlpW[...]
    b = mlpb[...]
    for j in range(k):
        sj = sg_ref[0, j]
        g1 = _leaky(sj + t)
        g2 = _leaky(_dot(g1, w) + b)
        acc = g2 if acc is None else jnp.maximum(acc, g2)
    o_ref[0] = _leaky(_dot(acc, pstW[...]) + pstb[...] + res_ref[0])


def _core_sc(sg, t, res, pr, k):
    B, _, M, dh = sg.shape
    d_out = pr['pstW'].shape[1]
    Mt = min(M, 128)
    mlpb = pr['mlpb'].reshape(1, dh)
    pstb = pr['pstb'].reshape(1, d_out)
    wspec = lambda shape: pl.BlockSpec(shape, lambda b, m: tuple(0 for _ in shape))
    return pl.pallas_call(
        partial(_core_sc_body, k=k),
        grid=(B, M // Mt),
        in_specs=[pl.BlockSpec((1, k, Mt, dh), lambda b, m: (b, 0, m, 0)),
                  pl.BlockSpec((1, Mt, dh), lambda b, m: (b, m, 0)),
                  pl.BlockSpec((1, Mt, d_out), lambda b, m: (b, m, 0)),
                  wspec((dh, dh)), wspec((1, dh)),
                  wspec((dh, d_out)), wspec((1, d_out))],
        out_specs=pl.BlockSpec((1, Mt, d_out), lambda b, m: (b, m, 0)),
        out_shape=jax.ShapeDtypeStruct((B, M, d_out), jnp.float32),
    )(sg, t, res, pr['mlpW'], mlpb, pr['pstW'], pstb)


def _stem_body(p_ref, W, b, o_ref):
    o_ref[0] = _leaky(_dot(p_ref[0], W[...]) + b[...])


def _stem(p, W, b):
    B, N, _ = p.shape
    D = W.shape[1]
    return pl.pallas_call(
        _stem_body,
        grid=(B,),
        in_specs=[pl.BlockSpec((1, N, 3), lambda i: (i, 0, 0)),
                  pl.BlockSpec((3, D), lambda i: (0, 0)),
                  pl.BlockSpec((1, D), lambda i: (0, 0))],
        out_specs=pl.BlockSpec((1, N, D), lambda i: (i, 0, 0)),
        out_shape=jax.ShapeDtypeStruct((B, N, D), jnp.float32),
    )(p, W, b.reshape(1, D))


def _head_body(f_ref, h1W, h1b, h2W, h2b, h3W, h3b, o_ref):
    f = f_ref[0]
    x = jnp.maximum(_dot(f, h1W[...]) + h1b[...], 0.0)
    mx = jnp.max(x, axis=0, keepdims=True)
    mn = jnp.mean(x, axis=0, keepdims=True)
    xc = jnp.concatenate([mx, mn], axis=1)
    x2 = jnp.maximum(_dot(xc, h2W[...]) + h2b[...], 0.0)
    o_ref[0] = _dot(x2, h3W[...]) + h3b[...]


def _head(f, hd):
    B, M, D = f.shape
    wspec = lambda shape: pl.BlockSpec(shape, lambda b: tuple(0 for _ in shape))
    out = pl.pallas_call(
        _head_body,
        grid=(B,),
        in_specs=[pl.BlockSpec((1, M, D), lambda b: (b, 0, 0)),
                  wspec((D, 1024)), wspec((1, 1024)),
                  wspec((2048, 512)), wspec((1, 512)),
                  wspec((512, 40)), wspec((1, 40))],
        out_specs=pl.BlockSpec((1, 1, 40), lambda b: (b, 0, 0)),
        out_shape=jax.ShapeDtypeStruct((B, 1, 40), jnp.float32),
    )(f, hd['h1W'], hd['h1b'].reshape(1, 1024), hd['h2W'],
      hd['h2b'].reshape(1, 512), hd['h3W'], hd['h3b'].reshape(1, 40))
    return out.reshape(B, 40)


def kernel(p, params):
    B = p.shape[0]
    f = _stem(p, params['stem']['W'], params['stem']['b'])
    pts = p
    idx = None
    gidx = None
    for pr, cfg in zip(params['cic'], _CFG):
        n_g, r_g, k_g, d_in, d_out, ratio = cfg
        M = n_g
        if pts.shape[1] != n_g:
            pce = _fps(pts, n_g)
            f = _ball(pts, pce, f, k_g, r_g)
            pts = pce
            idx = _knn(pts, k_g)
            gidx = (idx.transpose(0, 2, 1)
                    + (jnp.arange(B, dtype=jnp.int32) * M).reshape(B, 1, 1))
            gidx = gidx.reshape(1, B * k_g * M)
        s, t, res = _prep(f, pts, pr)
        dh = s.shape[2]
        if dh == 128:
            sg = _sc_gather(s.reshape(B * M, dh), gidx, dh)
            f = _core_sc(sg.reshape(B, k_g, M, dh), t, res, pr, k_g)
        else:
            f = _core(s, t, idx, res, pr, k_g)
    return _head(f, params['head'])


# speedup vs baseline: 1.4567x; 1.1365x over previous
import functools
from functools import partial

import jax
import jax.numpy as jnp
from jax.experimental import pallas as pl
from jax.experimental.pallas import tpu as pltpu
from jax.experimental.pallas import tpu_sc as plsc

_CFG = [
    (1024, 0.05, 20, 32, 64, 2),
    (1024, 0.05, 20, 64, 64, 4),
    (1024, 0.05, 20, 64, 128, 2),
    (1024, 0.1, 20, 128, 128, 4),
    (256, 0.1, 20, 128, 256, 2),
    (256, 0.2, 20, 256, 256, 4),
    (64, 0.2, 20, 256, 512, 2),
    (64, 0.4, 20, 512, 512, 4),
]


def _leaky(x):
    return jnp.where(x >= 0, x, 0.01 * x)


def _dot(a, b):
    return jnp.dot(a, b, preferred_element_type=jnp.float32)


def _fps_body(px_ref, py_ref, pz_ref, pce_ref, *, n, N, B):
    S = N // 128
    px = px_ref[...]
    py = py_ref[...]
    pz = pz_ref[...]
    row = jax.lax.broadcasted_iota(jnp.int32, (B, S, 128), 1)
    col = jax.lax.broadcasted_iota(jnp.int32, (B, S, 128), 2)
    flat = row * 128 + col

    def body(i, st):
        dist, far = st
        sel = flat == far
        cx = jnp.sum(jnp.where(sel, px, 0.0), axis=(1, 2), keepdims=True)
        cy = jnp.sum(jnp.where(sel, py, 0.0), axis=(1, 2), keepdims=True)
        cz = jnp.sum(jnp.where(sel, pz, 0.0), axis=(1, 2), keepdims=True)
        c = jnp.concatenate([cx, cy, cz], axis=2)
        pce_ref[:, pl.ds(i, 1), :] = c
        d = (px - cx) ** 2 + (py - cy) ** 2 + (pz - cz) ** 2
        dist = jnp.minimum(dist, d)
        mv = jnp.max(dist, axis=(1, 2), keepdims=True)
        far = jnp.min(jnp.where(dist == mv, flat, N), axis=(1, 2),
                      keepdims=True)
        return dist, far

    jax.lax.fori_loop(
        0, n, body,
        (jnp.full((B, S, 128), 1e10, jnp.float32),
         jnp.zeros((B, 1, 1), jnp.int32)))


def _fps(p, n):
    B, N, _ = p.shape
    S = N // 128
    pp = p.transpose(0, 2, 1).reshape(B, 3, S, 128)
    px, py, pz = pp[:, 0], pp[:, 1], pp[:, 2]
    full = lambda shape: pl.BlockSpec(shape, lambda g: tuple(0 for _ in shape))
    return pl.pallas_call(
        partial(_fps_body, n=n, N=N, B=B),
        grid=(1,),
        in_specs=[full((B, S, 128)), full((B, S, 128)), full((B, S, 128))],
        out_specs=full((B, n, 3)),
        out_shape=jax.ShapeDtypeStruct((B, n, 3), jnp.float32),
    )(px, py, pz)


def _ball_body(pq_ref, pkt_ref, f_ref, o_ref, *, k, r2, N):
    q = pq_ref[0]
    pk = pkt_ref[0]
    qq = jnp.sum(q * q, axis=1, keepdims=True)
    x, y, z = pk[0:1, :], pk[1:2, :], pk[2:3, :]
    pp = (x * x + y * y) + z * z
    cross = jnp.dot(q, pk, preferred_element_type=jnp.float32)
    sqr = (qq + pp) - 2.0 * cross
    Mt = q.shape[0]
    iota_n = jax.lax.broadcasted_iota(jnp.int32, (Mt, N), 1)
    key = jnp.where(sqr <= r2, iota_n, N)
    f2 = f_ref[0]
    G2 = f2.shape[0]
    P = N // G2
    D = f2.shape[1] // P
    iota_g = jax.lax.broadcasted_iota(jnp.int32, (Mt, G2), 1)
    iota_d = jax.lax.broadcasted_iota(jnp.int32, (Mt, D), 1)
    acc = jnp.full((Mt, D), -jnp.inf, jnp.float32)
    v0 = jnp.min(key, axis=1, keepdims=True)
    empty = v0 >= N

    def step(st):
        j, key, acc, v = st
        oh = (iota_g == v // P).astype(jnp.float32)
        wide = _dot(oh, f2)
        if P > 1:
            lane = (v % P) * D + iota_d
            fj = jnp.take_along_axis(wide, lane, axis=1)
        else:
            fj = wide
        acc = jnp.maximum(acc, jnp.where(v < N, fj, -jnp.inf))
        key = jnp.where(key == v, N, key)
        v = jnp.min(key, axis=1, keepdims=True)
        return j + 1, key, acc, v

    def cond(st):
        j, _, _, v = st
        return jnp.logical_and(j < k, jnp.min(v) < N)

    _, _, acc, _ = jax.lax.while_loop(
        cond, step, (jnp.int32(0), key, acc, v0))
    f_last = f2[G2 - 1:G2, (P - 1) * D:]
    o_ref[0] = jnp.where(empty, f_last, acc)


def _ball(p, pce, f, k, r):
    B, N, _ = p.shape
    M = pce.shape[1]
    D = f.shape[2]
    Mt = min(M, 512)
    P = max(1, 128 // D)
    f2 = f.reshape(B, N // P, P * D)
    pkt = p.transpose(0, 2, 1)
    return pl.pallas_call(
        partial(_ball_body, k=k, r2=r * r, N=N),
        grid=(B, M // Mt),
        in_specs=[pl.BlockSpec((1, Mt, 3), lambda b, m: (b, m, 0)),
                  pl.BlockSpec((1, 3, N), lambda b, m: (b, 0, 0)),
                  pl.BlockSpec((1, N // P, P * D), lambda b, m: (b, 0, 0))],
        out_specs=pl.BlockSpec((1, Mt, D), lambda b, m: (b, m, 0)),
        out_shape=jax.ShapeDtypeStruct((B, M, D), jnp.float32),
    )(pce, pkt, f2)


def _knn_body(pq_ref, pkt_ref, idx_ref, *, k, M):
    q = pq_ref[0]
    pk = pkt_ref[0]
    qq = jnp.sum(q * q, axis=1, keepdims=True)
    x, y, z = pk[0:1, :], pk[1:2, :], pk[2:3, :]
    pp = (x * x + y * y) + z * z
    cross = jnp.dot(q, pk, preferred_element_type=jnp.float32)
    key = (qq + pp) - 2.0 * cross
    Mt = q.shape[0]
    iota_m = jax.lax.broadcasted_iota(jnp.int32, (Mt, M), 1)
    cols = []
    for _ in range(k):
        v = jnp.min(key, axis=1, keepdims=True)
        nj = jnp.min(jnp.where(key == v, iota_m, M), axis=1, keepdims=True)
        cols.append(nj)
        key = jnp.where(iota_m == nj, jnp.inf, key)
    idx_ref[0] = jnp.concatenate(cols, axis=1)


def _knn(p, k):
    B, M, _ = p.shape
    Mt = min(M, 512)
    pkt = p.transpose(0, 2, 1)
    return pl.pallas_call(
        partial(_knn_body, k=k, M=M),
        grid=(B, M // Mt),
        in_specs=[pl.BlockSpec((1, Mt, 3), lambda b, m: (b, m, 0)),
                  pl.BlockSpec((1, 3, M), lambda b, m: (b, 0, 0))],
        out_specs=pl.BlockSpec((1, Mt, k), lambda b, m: (b, m, 0)),
        out_shape=jax.ShapeDtypeStruct((B, M, k), jnp.int32),
    )(p, pkt)


def _prep_body(f_ref, p_ref, preW, preb, A, Bw, p2fb, s_ref, t_ref):
    f = f_ref[0]
    p_ = p_ref[0]
    fpre = _leaky(_dot(f, preW[...]) + preb[...])
    s_ref[0] = fpre + _dot(p_, Bw[...])
    t_ref[0] = _dot(p_, A[...]) + p2fb[...] - fpre


def _prep_sc_body(f_ref, p_ref, preW, preb, A, Bw, p2fb, scW, scb,
                  s_ref, t_ref, res_ref):
    f = f_ref[0]
    p_ = p_ref[0]
    fpre = _leaky(_dot(f, preW[...]) + preb[...])
    s_ref[0] = fpre + _dot(p_, Bw[...])
    t_ref[0] = _dot(p_, A[...]) + p2fb[...] - fpre
    res_ref[0] = _dot(f, scW[...]) + scb[...]


def _prep(f, pts, pr):
    B, M, d_in = f.shape
    dh = pr['preW'].shape[1]
    p2fW = pr['p2fW']
    A = p2fW[0:3] - p2fW[6:9]
    Bw = p2fW[3:6] + p2fW[6:9]
    preb = pr['preb'].reshape(1, dh)
    p2fb = pr['p2fb'].reshape(1, dh)
    wspec = lambda shape: pl.BlockSpec(shape, lambda b: tuple(0 for _ in shape))
    has_sc = 'scW' in pr
    if has_sc:
        d_out = pr['scW'].shape[1]
        scb = pr['scb'].reshape(1, d_out)
        out_shapes = [jax.ShapeDtypeStruct((B, M, dh), jnp.float32),
                      jax.ShapeDtypeStruct((B, M, dh), jnp.float32),
                      jax.ShapeDtypeStruct((B, M, d_out), jnp.float32)]
        out_specs = [pl.BlockSpec((1, M, dh), lambda b: (b, 0, 0)),
                     pl.BlockSpec((1, M, dh), lambda b: (b, 0, 0)),
                     pl.BlockSpec((1, M, d_out), lambda b: (b, 0, 0))]
        s, t, res = pl.pallas_call(
            _prep_sc_body,
            grid=(B,),
            in_specs=[pl.BlockSpec((1, M, d_in), lambda b: (b, 0, 0)),
                      pl.BlockSpec((1, M, 3), lambda b: (b, 0, 0)),
                      wspec((d_in, dh)), wspec((1, dh)),
                      wspec((3, dh)), wspec((3, dh)), wspec((1, dh)),
                      wspec((d_in, d_out)), wspec((1, d_out))],
            out_specs=out_specs,
            out_shape=out_shapes,
        )(f, pts, pr['preW'], preb, A, Bw, p2fb, pr['scW'], scb)
        return s, t, res
    s, t = pl.pallas_call(
        _prep_body,
        grid=(B,),
        in_specs=[pl.BlockSpec((1, M, d_in), lambda b: (b, 0, 0)),
                  pl.BlockSpec((1, M, 3), lambda b: (b, 0, 0)),
                  wspec((d_in, dh)), wspec((1, dh)),
                  wspec((3, dh)), wspec((3, dh)), wspec((1, dh))],
        out_specs=[pl.BlockSpec((1, M, dh), lambda b: (b, 0, 0)),
                   pl.BlockSpec((1, M, dh), lambda b: (b, 0, 0))],
        out_shape=[jax.ShapeDtypeStruct((B, M, dh), jnp.float32),
                   jax.ShapeDtypeStruct((B, M, dh), jnp.float32)],
    )(f, pts, pr['preW'], preb, A, Bw, p2fb)
    return s, t, f


def _sc_gather(s_flat, gidx, dh):
    G = gidx.shape[1]
    window = 128
    mesh = plsc.VectorSubcoreMesh(core_axis_name="core",
                                  subcore_axis_name="subcore")

    @partial(pl.kernel,
             out_type=jax.ShapeDtypeStruct((G, dh), jnp.float32),
             mesh=mesh)
    def kern(x_hbm, i_hbm, o_hbm):
        def body(i_vmem, o_vmem):
            pltpu.sync_copy(x_hbm.at[i_vmem.at[0]], o_vmem)

        pltpu.emit_pipeline(
            body,
            grid=(G // window,),
            in_specs=[pl.BlockSpec((1, window), lambda i: (0, i))],
            out_specs=[pl.BlockSpec((window, dh), lambda i: (i, 0))],
            core_axis_name=("core", "subcore"),
            dimension_semantics=(pltpu.PARALLEL,),
        )(i_hbm, o_hbm)

    return kern(s_flat, gidx)


def _core_body(s2_ref, t_ref, idx_ref, res_ref, mlpW, mlpb, pstW, pstb,
               o_ref, *, k, P, dh):
    s2 = s2_ref[0]
    t = t_ref[0]
    idx = idx_ref[0]
    Mt = t.shape[0]
    G2 = s2.shape[0]
    iota_g = jax.lax.broadcasted_iota(jnp.int32, (Mt, G2), 1)
    iota_d = jax.lax.broadcasted_iota(jnp.int32, (Mt, dh), 1)
    acc = None
    w = mlpW[...]
    b = mlpb[...]
    for j in range(k):
        ij = idx[:, j:j + 1]
        oh = (iota_g == ij // P).astype(jnp.float32)
        wide = _dot(oh, s2)
        if P > 1:
            lane = (ij % P) * dh + iota_d
            sj = jnp.take_along_axis(wide, lane, axis=1)
        else:
            sj = wide
        g1 = _leaky(sj + t)
        g2 = _leaky(_dot(g1, w) + b)
        acc = g2 if acc is None else jnp.maximum(acc, g2)
    o_ref[0] = _leaky(_dot(acc, pstW[...]) + pstb[...] + res_ref[0])


def _core(s, t, idx, res, pr, k):
    B, M, dh = s.shape
    d_out = pr['pstW'].shape[1]
    Mt = min(M, 512)
    P = max(1, 128 // dh)
    s2 = s.reshape(B, M // P, P * dh)
    mlpb = pr['mlpb'].reshape(1, dh)
    pstb = pr['pstb'].reshape(1, d_out)
    wspec = lambda shape: pl.BlockSpec(shape, lambda b, m: tuple(0 for _ in shape))
    return pl.pallas_call(
        partial(_core_body, k=k, P=P, dh=dh),
        grid=(B, M // Mt),
        in_specs=[pl.BlockSpec((1, M // P, P * dh), lambda b, m: (b, 0, 0)),
                  pl.BlockSpec((1, Mt, dh), lambda b, m: (b, m, 0)),
                  pl.BlockSpec((1, Mt, k), lambda b, m: (b, m, 0)),
                  pl.BlockSpec((1, Mt, d_out), lambda b, m: (b, m, 0)),
                  wspec((dh, dh)), wspec((1, dh)),
                  wspec((dh, d_out)), wspec((1, d_out))],
        out_specs=pl.BlockSpec((1, Mt, d_out), lambda b, m: (b, m, 0)),
        out_shape=jax.ShapeDtypeStruct((B, M, d_out), jnp.float32),
    )(s2, t, idx, res, pr['mlpW'], mlpb, pr['pstW'], pstb)


def _core_sc_body(sg_ref, t_ref, res_ref, mlpW, mlpb, pstW, pstb,
                  o_ref, *, k):
    t = t_ref[0]
    acc = None
    w = mlpW[...]
    b = mlpb[...]
    for j in range(k):
        sj = sg_ref[0, j]
        g1 = _leaky(sj + t)
        g2 = _leaky(_dot(g1, w) + b)
        acc = g2 if acc is None else jnp.maximum(acc, g2)
    o_ref[0] = _leaky(_dot(acc, pstW[...]) + pstb[...] + res_ref[0])


def _core_sc(sg, t, res, pr, k):
    B, _, M, dh = sg.shape
    d_out = pr['pstW'].shape[1]
    Mt = min(M, 128)
    mlpb = pr['mlpb'].reshape(1, dh)
    pstb = pr['pstb'].reshape(1, d_out)
    wspec = lambda shape: pl.BlockSpec(shape, lambda b, m: tuple(0 for _ in shape))
    return pl.pallas_call(
        partial(_core_sc_body, k=k),
        grid=(B, M // Mt),
        in_specs=[pl.BlockSpec((1, k, Mt, dh), lambda b, m: (b, 0, m, 0)),
                  pl.BlockSpec((1, Mt, dh), lambda b, m: (b, m, 0)),
                  pl.BlockSpec((1, Mt, d_out), lambda b, m: (b, m, 0)),
                  wspec((dh, dh)), wspec((1, dh)),
                  wspec((dh, d_out)), wspec((1, d_out))],
        out_specs=pl.BlockSpec((1, Mt, d_out), lambda b, m: (b, m, 0)),
        out_shape=jax.ShapeDtypeStruct((B, M, d_out), jnp.float32),
    )(sg, t, res, pr['mlpW'], mlpb, pr['pstW'], pstb)


def _stem_body(p_ref, W, b, o_ref):
    o_ref[0] = _leaky(_dot(p_ref[0], W[...]) + b[...])


def _stem(p, W, b):
    B, N, _ = p.shape
    D = W.shape[1]
    return pl.pallas_call(
        _stem_body,
        grid=(B,),
        in_specs=[pl.BlockSpec((1, N, 3), lambda i: (i, 0, 0)),
                  pl.BlockSpec((3, D), lambda i: (0, 0)),
                  pl.BlockSpec((1, D), lambda i: (0, 0))],
        out_specs=pl.BlockSpec((1, N, D), lambda i: (i, 0, 0)),
        out_shape=jax.ShapeDtypeStruct((B, N, D), jnp.float32),
    )(p, W, b.reshape(1, D))


def _head_body(f_ref, h1W, h1b, h2W, h2b, h3W, h3b, o_ref):
    f = f_ref[0]
    x = jnp.maximum(_dot(f, h1W[...]) + h1b[...], 0.0)
    mx = jnp.max(x, axis=0, keepdims=True)
    mn = jnp.mean(x, axis=0, keepdims=True)
    xc = jnp.concatenate([mx, mn], axis=1)
    x2 = jnp.maximum(_dot(xc, h2W[...]) + h2b[...], 0.0)
    o_ref[0] = _dot(x2, h3W[...]) + h3b[...]


def _head(f, hd):
    B, M, D = f.shape
    wspec = lambda shape: pl.BlockSpec(shape, lambda b: tuple(0 for _ in shape))
    out = pl.pallas_call(
        _head_body,
        grid=(B,),
        in_specs=[pl.BlockSpec((1, M, D), lambda b: (b, 0, 0)),
                  wspec((D, 1024)), wspec((1, 1024)),
                  wspec((2048, 512)), wspec((1, 512)),
                  wspec((512, 40)), wspec((1, 40))],
        out_specs=pl.BlockSpec((1, 1, 40), lambda b: (b, 0, 0)),
        out_shape=jax.ShapeDtypeStruct((B, 1, 40), jnp.float32),
    )(f, hd['h1W'], hd['h1b'].reshape(1, 1024), hd['h2W'],
      hd['h2b'].reshape(1, 512), hd['h3W'], hd['h3b'].reshape(1, 40))
    return out.reshape(B, 40)


def kernel(p, params):
    B = p.shape[0]
    f = _stem(p, params['stem']['W'], params['stem']['b'])
    pts = p
    idx = None
    gidx = None
    for pr, cfg in zip(params['cic'], _CFG):
        n_g, r_g, k_g, d_in, d_out, ratio = cfg
        M = n_g
        if pts.shape[1] != n_g:
            pce = _fps(pts, n_g)
            f = _ball(pts, pce, f, k_g, r_g)
            pts = pce
            idx = _knn(pts, k_g)
            gidx = (idx.transpose(0, 2, 1)
                    + (jnp.arange(B, dtype=jnp.int32) * M).reshape(B, 1, 1))
            gidx = gidx.reshape(1, B * k_g * M)
        s, t, res = _prep(f, pts, pr)
        dh = s.shape[2]
        if dh == 128:
            sg = _sc_gather(s.reshape(B * M, dh), gidx, dh)
            f = _core_sc(sg.reshape(B, k_g, M, dh), t, res, pr, k_g)
        else:
            f = _core(s, t, idx, res, pr, k_g)
    return _head(f, params['head'])
